# baseline (speedup 1.0000x reference)
import functools

import jax
import jax.numpy as jnp
from jax import lax
from jax.experimental import pallas as pl
from jax.experimental.pallas import tpu as pltpu
from jax.experimental.pallas import tpu_sc as plsc

_N = 10000
_E = 320000
_D = 128
_DZ = 64

_NC = 2
_NS = 16
_NW = _NC * _NS
_CH = 96
_NPH = 3
_PH = 36
_K = 12
_NCH = _NPH * _PH
_EPAD = _NW * _NCH * _CH
_RPT = 632
_NACC = _NS * _RPT
_RPT_D = 640
_NACC_D = _NS * _RPT_D


def _sc_mesh():
    return plsc.VectorSubcoreMesh(core_axis_name="c", subcore_axis_name="s")


@functools.partial(
    pl.kernel,
    mesh=_sc_mesh(),
    out_type=jax.ShapeDtypeStruct((_NC * _NACC_D,), jnp.float32),
    scratch_types=[
        pltpu.VMEM((_NPH, _PH, _CH), jnp.int32),
        pltpu.VMEM((_CH,), jnp.float32),
        pltpu.VMEM_SHARED((_NACC_D,), jnp.float32),
        pltpu.SemaphoreType.DMA,
    ],
)
def _deg_kernel(dstr_hbm, zeros1_hbm, out_hbm, dst_v, ones_v, acc_sh, sem):
    c = lax.axis_index("c")
    s = lax.axis_index("s")
    w = c * _NS + s
    for i in range(_CH // 16):
        ones_v[pl.ds(i * 16, 16)] = jnp.ones((16,), jnp.float32)
    pltpu.sync_copy(zeros1_hbm.at[pl.ds(s * _RPT_D, _RPT_D)],
                    acc_sh.at[pl.ds(s * _RPT_D, _RPT_D)])
    pltpu.sync_copy(dstr_hbm.at[w], dst_v)
    plsc.subcore_barrier()

    for h in range(_NPH):
        def body(p, carry):
            pltpu.sync_copy(ones_v, acc_sh.at[dst_v.at[h, p]], add=True)
            return carry

        lax.fori_loop(0, _PH, body, 0)
    plsc.subcore_barrier()
    pltpu.sync_copy(acc_sh.at[pl.ds(s * _RPT_D, _RPT_D)],
                    out_hbm.at[pl.ds(c * _NACC_D + s * _RPT_D, _RPT_D)])


@functools.partial(
    pl.kernel,
    mesh=_sc_mesh(),
    out_type=jax.ShapeDtypeStruct((_NC, _NACC, _D), jnp.float32),
    scratch_types=[
        pltpu.VMEM((_PH, _CH), jnp.int32),
        pltpu.VMEM((_PH, _CH), jnp.int32),
        pltpu.VMEM((3, _CH, _D), jnp.float32),
        pltpu.VMEM_SHARED((_NACC, _D), jnp.float32),
        pltpu.SemaphoreType.DMA,
        pltpu.SemaphoreType.DMA,
        pltpu.SemaphoreType.DMA,
    ],
)
def _agg_kernel(hs_hbm, srcr_hbm, dstr_hbm, zeros_hbm, out_hbm,
                src_v, dst_v, rows_v, acc_sh, gs0, gs1, gs2):
    c = lax.axis_index("c")
    s = lax.axis_index("s")
    w = c * _NS + s
    gsems = (gs0, gs1, gs2)
    pltpu.sync_copy(zeros_hbm.at[pl.ds(s * _RPT, _RPT)],
                    acc_sh.at[pl.ds(s * _RPT, _RPT)])
    plsc.subcore_barrier()

    for h in range(_NPH):
        pltpu.sync_copy(srcr_hbm.at[w, h], src_v)
        pltpu.sync_copy(dstr_hbm.at[w, h], dst_v)

        def body(p, carry):
            base = p * _K

            def fire_g(jj):
                return pltpu.async_copy(hs_hbm.at[src_v.at[base + jj]],
                                        rows_v.at[jj % 3], gsems[jj % 3])

            def scat(jj):
                pltpu.sync_copy(rows_v.at[jj % 3],
                                acc_sh.at[dst_v.at[base + jj]], add=True)

            gh = {0: fire_g(0), 1: fire_g(1)}
            for jj in range(_K):
                gh[jj].wait()
                scat(jj)
                if jj + 2 < _K:
                    gh[jj + 2] = fire_g(jj + 2)
            return carry

        lax.fori_loop(0, _PH // _K, body, 0)
    plsc.subcore_barrier()
    pltpu.sync_copy(acc_sh.at[pl.ds(s * _RPT, _RPT)],
                    out_hbm.at[c, pl.ds(s * _RPT, _RPT)])


_RB = 1000


def _m1_body(x_ref, w_ref, dinv_ref, o_ref):
    h = jnp.dot(x_ref[...], w_ref[...], preferred_element_type=jnp.float32)
    o_ref[...] = dinv_ref[...] * h


def _m1(x, W1, dinv2):
    return pl.pallas_call(
        _m1_body,
        grid=(_N // _RB,),
        in_specs=[
            pl.BlockSpec((_RB, _D), lambda i: (i, 0)),
            pl.BlockSpec((_D, _D), lambda i: (0, 0)),
            pl.BlockSpec((_RB, 1), lambda i: (i, 0)),
        ],
        out_specs=pl.BlockSpec((_RB, _D), lambda i: (i, 0)),
        out_shape=jax.ShapeDtypeStruct((_N, _D), jnp.float32),
    )(x, W1, dinv2)


def _m2_body(a_ref, hs_ref, dinv_ref, b_ref, w_ref, o_ref):
    t = dinv_ref[...] * (a_ref[0] + a_ref[1] + hs_ref[...]) + b_ref[...]
    h = jnp.maximum(t, 0.0)
    o_ref[...] = dinv_ref[...] * jnp.dot(
        h, w_ref[...], preferred_element_type=jnp.float32)


def _m2(agg, hs, dinv2, b, W):
    return pl.pallas_call(
        _m2_body,
        grid=(_N // _RB,),
        in_specs=[
            pl.BlockSpec((2, _RB, _D), lambda i: (0, i, 0)),
            pl.BlockSpec((_RB, _D), lambda i: (i, 0)),
            pl.BlockSpec((_RB, 1), lambda i: (i, 0)),
            pl.BlockSpec((1, _D), lambda i: (0, 0)),
            pl.BlockSpec((_D, _D), lambda i: (0, 0)),
        ],
        out_specs=pl.BlockSpec((_RB, _D), lambda i: (i, 0)),
        out_shape=jax.ShapeDtypeStruct((_N, _D), jnp.float32),
    )(agg, hs, dinv2, b, W)


def _m3_body(a_ref, hs_ref, dinv_ref, b_ref, o_ref):
    t = dinv_ref[...] * (a_ref[0] + a_ref[1] + hs_ref[...]) + b_ref[...]
    o_ref[...] = dinv_ref[...] * jnp.maximum(t, 0.0)


def _m3(agg, hs, dinv2, b):
    return pl.pallas_call(
        _m3_body,
        grid=(_N // _RB,),
        in_specs=[
            pl.BlockSpec((2, _RB, _D), lambda i: (0, i, 0)),
            pl.BlockSpec((_RB, _D), lambda i: (i, 0)),
            pl.BlockSpec((_RB, 1), lambda i: (i, 0)),
            pl.BlockSpec((1, _D), lambda i: (0, 0)),
        ],
        out_specs=pl.BlockSpec((_RB, _D), lambda i: (i, 0)),
        out_shape=jax.ShapeDtypeStruct((_N, _D), jnp.float32),
    )(agg, hs, dinv2, b)


def _m4_body(a_ref, hs_ref, dinv_ref, wmu_ref, bmu_ref, wlv_ref, blv_ref,
             mu_ref, lv_ref):
    aggf = dinv_ref[...] * (a_ref[0] + a_ref[1] + hs_ref[...])
    mu_ref[...] = jnp.dot(aggf, wmu_ref[...],
                          preferred_element_type=jnp.float32) + bmu_ref[...]
    lv_ref[...] = jnp.dot(aggf, wlv_ref[...],
                          preferred_element_type=jnp.float32) + blv_ref[...]


def _m4(agg, hs, dinv2, Wmu, bmu, Wlv, blv):
    return pl.pallas_call(
        _m4_body,
        grid=(_N // _RB,),
        in_specs=[
            pl.BlockSpec((2, _RB, _D), lambda i: (0, i, 0)),
            pl.BlockSpec((_RB, _D), lambda i: (i, 0)),
            pl.BlockSpec((_RB, 1), lambda i: (i, 0)),
            pl.BlockSpec((_D, _DZ), lambda i: (0, 0)),
            pl.BlockSpec((1, _DZ), lambda i: (0, 0)),
            pl.BlockSpec((_D, _DZ), lambda i: (0, 0)),
            pl.BlockSpec((1, _DZ), lambda i: (0, 0)),
        ],
        out_specs=[
            pl.BlockSpec((_RB, _DZ), lambda i: (i, 0)),
            pl.BlockSpec((_RB, _DZ), lambda i: (i, 0)),
        ],
        out_shape=[
            jax.ShapeDtypeStruct((_N, _DZ), jnp.float32),
            jax.ShapeDtypeStruct((_N, _DZ), jnp.float32),
        ],
    )(agg, hs, dinv2, Wmu, bmu, Wlv, blv)


def kernel(x, edge_index, W1, b1, W2, b2, Wmu, bmu, Wlv, blv):
    src = edge_index[0]
    dst = edge_index[1]
    pad = _EPAD - _E
    srcr = jnp.concatenate(
        [src, jnp.zeros((pad,), jnp.int32)]).reshape(_NW, _NPH, _PH, _CH)
    dstr = jnp.concatenate(
        [dst, jnp.full((pad,), _N, jnp.int32)]).reshape(_NW, _NPH, _PH, _CH)
    zeros1 = jnp.zeros((_NACC_D,), jnp.float32)
    zeros2 = jnp.zeros((_NACC, _D), jnp.float32)

    degp = _deg_kernel(dstr, zeros1).reshape(_NC, _NACC_D)
    deg = degp[0, :_N] + degp[1, :_N] + 1.0
    dinv2 = lax.rsqrt(deg)[:, None]

    b1r = b1[None, :]
    b2r = b2[None, :]
    bmur = bmu[None, :]
    blvr = blv[None, :]

    hs1 = _m1(x, W1, dinv2)
    agg1 = _agg_kernel(hs1, srcr, dstr, zeros2)
    hs2 = _m2(agg1, hs1, dinv2, b1r, W2)
    agg2 = _agg_kernel(hs2, srcr, dstr, zeros2)
    hs3 = _m3(agg2, hs2, dinv2, b2r)
    agg3 = _agg_kernel(hs3, srcr, dstr, zeros2)
    mu, logvar = _m4(agg3, hs3, dinv2, Wmu, bmur, Wlv, blvr)
    return (mu, mu, logvar)


# speedup vs baseline: 1.0772x; 1.0772x over previous
import functools

import jax
import jax.numpy as jnp
from jax import lax
from jax.experimental import pallas as pl
from jax.experimental.pallas import tpu as pltpu
from jax.experimental.pallas import tpu_sc as plsc

_N = 10000
_E = 320000
_D = 128
_DZ = 64

_NC = 2
_NS = 16
_NW = _NC * _NS
_CH = 128
_NCH = 80
_EPAD = _NW * _NCH * _CH
_RPT = 640
_NACC = _NS * _RPT


def _sc_mesh():
    return plsc.VectorSubcoreMesh(core_axis_name="c", subcore_axis_name="s")


@functools.partial(
    pl.kernel,
    mesh=_sc_mesh(),
    out_type=jax.ShapeDtypeStruct((_NC * _NACC,), jnp.float32),
    scratch_types=[
        pltpu.VMEM((_NCH, _CH), jnp.int32),
        pltpu.VMEM((_CH,), jnp.float32),
        pltpu.VMEM_SHARED((_NACC,), jnp.float32),
    ],
)
def _deg_kernel(dstr_hbm, zeros1_hbm, out_hbm, dst_v, ones_v, acc_sh):
    c = lax.axis_index("c")
    s = lax.axis_index("s")
    w = c * _NS + s
    for i in range(_CH // 16):
        ones_v[pl.ds(i * 16, 16)] = jnp.ones((16,), jnp.float32)
    pltpu.sync_copy(zeros1_hbm.at[pl.ds(s * _RPT, _RPT)],
                    acc_sh.at[pl.ds(s * _RPT, _RPT)])
    pltpu.sync_copy(dstr_hbm.at[w], dst_v)
    plsc.subcore_barrier()

    def body(j, carry):
        pltpu.sync_copy(ones_v, acc_sh.at[dst_v.at[j]], add=True)
        return carry

    lax.fori_loop(0, _NCH, body, 0)
    plsc.subcore_barrier()
    pltpu.sync_copy(acc_sh.at[pl.ds(s * _RPT, _RPT)],
                    out_hbm.at[pl.ds(c * _NACC + s * _RPT, _RPT)])


@functools.partial(
    pl.kernel,
    mesh=_sc_mesh(),
    out_type=jax.ShapeDtypeStruct((_NC, _NACC, _D), jnp.float32),
    scratch_types=[
        pltpu.VMEM((_NCH, _CH), jnp.int32),
        pltpu.VMEM((_NCH, _CH), jnp.int32),
        pltpu.VMEM((_CH, _D), jnp.float32),
        pltpu.VMEM_SHARED((_NACC, _D), jnp.float32),
        pltpu.SemaphoreType.DMA,
    ],
)
def _agg_kernel(hs_hbm, srcr_hbm, dstr_hbm, zeros_hbm, out_hbm,
                src_v, dst_v, rows_v, acc_sh, sem):
    c = lax.axis_index("c")
    s = lax.axis_index("s")
    w = c * _NS + s
    pltpu.sync_copy(zeros_hbm.at[pl.ds(s * _RPT, _RPT)],
                    acc_sh.at[pl.ds(s * _RPT, _RPT)])
    pltpu.sync_copy(srcr_hbm.at[w], src_v)
    pltpu.sync_copy(dstr_hbm.at[w], dst_v)
    plsc.subcore_barrier()

    def body(j, carry):
        pltpu.async_copy(hs_hbm.at[src_v.at[j]], rows_v, sem).wait()
        pltpu.sync_copy(rows_v, acc_sh.at[dst_v.at[j]], add=True)
        return carry

    lax.fori_loop(0, _NCH, body, 0)
    plsc.subcore_barrier()
    pltpu.sync_copy(acc_sh.at[pl.ds(s * _RPT, _RPT)],
                    out_hbm.at[c, pl.ds(s * _RPT, _RPT)])


_RB = 1000


def _m1_body(x_ref, w_ref, dinv_ref, o_ref):
    h = jnp.dot(x_ref[...], w_ref[...], preferred_element_type=jnp.float32)
    o_ref[...] = dinv_ref[...] * h


def _m1(x, W1, dinv2):
    return pl.pallas_call(
        _m1_body,
        grid=(_N // _RB,),
        in_specs=[
            pl.BlockSpec((_RB, _D), lambda i: (i, 0)),
            pl.BlockSpec((_D, _D), lambda i: (0, 0)),
            pl.BlockSpec((_RB, 1), lambda i: (i, 0)),
        ],
        out_specs=pl.BlockSpec((_RB, _D), lambda i: (i, 0)),
        out_shape=jax.ShapeDtypeStruct((_N, _D), jnp.float32),
    )(x, W1, dinv2)


def _m2_body(a_ref, hs_ref, dinv_ref, b_ref, w_ref, o_ref):
    t = dinv_ref[...] * (a_ref[0] + a_ref[1] + hs_ref[...]) + b_ref[...]
    h = jnp.maximum(t, 0.0)
    o_ref[...] = dinv_ref[...] * jnp.dot(
        h, w_ref[...], preferred_element_type=jnp.float32)


def _m2(agg, hs, dinv2, b, W):
    return pl.pallas_call(
        _m2_body,
        grid=(_N // _RB,),
        in_specs=[
            pl.BlockSpec((2, _RB, _D), lambda i: (0, i, 0)),
            pl.BlockSpec((_RB, _D), lambda i: (i, 0)),
            pl.BlockSpec((_RB, 1), lambda i: (i, 0)),
            pl.BlockSpec((1, _D), lambda i: (0, 0)),
            pl.BlockSpec((_D, _D), lambda i: (0, 0)),
        ],
        out_specs=pl.BlockSpec((_RB, _D), lambda i: (i, 0)),
        out_shape=jax.ShapeDtypeStruct((_N, _D), jnp.float32),
    )(agg, hs, dinv2, b, W)


def _m3_body(a_ref, hs_ref, dinv_ref, b_ref, o_ref):
    t = dinv_ref[...] * (a_ref[0] + a_ref[1] + hs_ref[...]) + b_ref[...]
    o_ref[...] = dinv_ref[...] * jnp.maximum(t, 0.0)


def _m3(agg, hs, dinv2, b):
    return pl.pallas_call(
        _m3_body,
        grid=(_N // _RB,),
        in_specs=[
            pl.BlockSpec((2, _RB, _D), lambda i: (0, i, 0)),
            pl.BlockSpec((_RB, _D), lambda i: (i, 0)),
            pl.BlockSpec((_RB, 1), lambda i: (i, 0)),
            pl.BlockSpec((1, _D), lambda i: (0, 0)),
        ],
        out_specs=pl.BlockSpec((_RB, _D), lambda i: (i, 0)),
        out_shape=jax.ShapeDtypeStruct((_N, _D), jnp.float32),
    )(agg, hs, dinv2, b)


def _m4_body(a_ref, hs_ref, dinv_ref, wmu_ref, bmu_ref, wlv_ref, blv_ref,
             mu_ref, lv_ref):
    aggf = dinv_ref[...] * (a_ref[0] + a_ref[1] + hs_ref[...])
    mu_ref[...] = jnp.dot(aggf, wmu_ref[...],
                          preferred_element_type=jnp.float32) + bmu_ref[...]
    lv_ref[...] = jnp.dot(aggf, wlv_ref[...],
                          preferred_element_type=jnp.float32) + blv_ref[...]


def _m4(agg, hs, dinv2, Wmu, bmu, Wlv, blv):
    return pl.pallas_call(
        _m4_body,
        grid=(_N // _RB,),
        in_specs=[
            pl.BlockSpec((2, _RB, _D), lambda i: (0, i, 0)),
            pl.BlockSpec((_RB, _D), lambda i: (i, 0)),
            pl.BlockSpec((_RB, 1), lambda i: (i, 0)),
            pl.BlockSpec((_D, _DZ), lambda i: (0, 0)),
            pl.BlockSpec((1, _DZ), lambda i: (0, 0)),
            pl.BlockSpec((_D, _DZ), lambda i: (0, 0)),
            pl.BlockSpec((1, _DZ), lambda i: (0, 0)),
        ],
        out_specs=[
            pl.BlockSpec((_RB, _DZ), lambda i: (i, 0)),
            pl.BlockSpec((_RB, _DZ), lambda i: (i, 0)),
        ],
        out_shape=[
            jax.ShapeDtypeStruct((_N, _DZ), jnp.float32),
            jax.ShapeDtypeStruct((_N, _DZ), jnp.float32),
        ],
    )(agg, hs, dinv2, Wmu, bmu, Wlv, blv)


def kernel(x, edge_index, W1, b1, W2, b2, Wmu, bmu, Wlv, blv):
    src = edge_index[0]
    dst = edge_index[1]
    pad = _EPAD - _E
    pad_dst = _N + (jnp.arange(pad, dtype=jnp.int32) % (_NACC - _N))
    srcr = jnp.concatenate(
        [src, jnp.zeros((pad,), jnp.int32)]).reshape(_NW, _NCH, _CH)
    dstr = jnp.concatenate([dst, pad_dst]).reshape(_NW, _NCH, _CH)
    zeros1 = jnp.zeros((_NACC,), jnp.float32)
    zeros2 = jnp.zeros((_NACC, _D), jnp.float32)

    degp = _deg_kernel(dstr, zeros1).reshape(_NC, _NACC)
    deg = degp[0, :_N] + degp[1, :_N] + 1.0
    dinv2 = lax.rsqrt(deg)[:, None]

    b1r = b1[None, :]
    b2r = b2[None, :]
    bmur = bmu[None, :]
    blvr = blv[None, :]

    hs1 = _m1(x, W1, dinv2)
    agg1 = _agg_kernel(hs1, srcr, dstr, zeros2)
    hs2 = _m2(agg1, hs1, dinv2, b1r, W2)
    agg2 = _agg_kernel(hs2, srcr, dstr, zeros2)
    hs3 = _m3(agg2, hs2, dinv2, b2r)
    agg3 = _agg_kernel(hs3, srcr, dstr, zeros2)
    mu, logvar = _m4(agg3, hs3, dinv2, Wmu, bmur, Wlv, blvr)
    return (mu, mu, logvar)


# speedup vs baseline: 1.1858x; 1.1008x over previous
import functools

import jax
import jax.numpy as jnp
from jax import lax
from jax.experimental import pallas as pl
from jax.experimental.pallas import tpu as pltpu
from jax.experimental.pallas import tpu_sc as plsc

_N = 10000
_E = 320000
_D = 128
_DZ = 64

_NC = 2
_NS = 16
_NW = _NC * _NS
_CH = 128
_NCH = 80
_EPAD = _NW * _NCH * _CH
_RPT = 640
_NACC = _NS * _RPT


def _sc_mesh():
    return plsc.VectorSubcoreMesh(core_axis_name="c", subcore_axis_name="s")


@functools.partial(
    pl.kernel,
    mesh=_sc_mesh(),
    out_type=jax.ShapeDtypeStruct((_NC * _NACC,), jnp.float32),
    scratch_types=[
        pltpu.VMEM((_NCH, _CH), jnp.int32),
        pltpu.VMEM((_CH,), jnp.float32),
        pltpu.VMEM_SHARED((_NACC,), jnp.float32),
    ],
)
def _deg_kernel(dstr_hbm, zeros1_hbm, out_hbm, dst_v, ones_v, acc_sh):
    c = lax.axis_index("c")
    s = lax.axis_index("s")
    w = c * _NS + s
    for i in range(_CH // 16):
        ones_v[pl.ds(i * 16, 16)] = jnp.ones((16,), jnp.float32)
    pltpu.sync_copy(zeros1_hbm.at[pl.ds(s * _RPT, _RPT)],
                    acc_sh.at[pl.ds(s * _RPT, _RPT)])
    pltpu.sync_copy(dstr_hbm.at[w], dst_v)
    plsc.subcore_barrier()

    def body(j, carry):
        pltpu.sync_copy(ones_v, acc_sh.at[dst_v.at[j]], add=True)
        return carry

    lax.fori_loop(0, _NCH, body, 0)
    plsc.subcore_barrier()
    pltpu.sync_copy(acc_sh.at[pl.ds(s * _RPT, _RPT)],
                    out_hbm.at[pl.ds(c * _NACC + s * _RPT, _RPT)])


@functools.partial(
    pl.kernel,
    mesh=_sc_mesh(),
    out_type=jax.ShapeDtypeStruct((_NC, _NACC, _D), jnp.float32),
    scratch_types=[
        pltpu.VMEM((_NCH, _CH), jnp.int32),
        pltpu.VMEM((_NCH, _CH), jnp.int32),
        pltpu.VMEM((_CH, _D), jnp.float32),
        pltpu.VMEM_SHARED((_NACC, _D), jnp.float32),
        pltpu.SemaphoreType.DMA,
    ],
)
def _agg_kernel(hs_hbm, srcr_hbm, dstr_hbm, zeros_hbm, out_hbm,
                src_v, dst_v, rows_v, acc_sh, sem):
    c = lax.axis_index("c")
    s = lax.axis_index("s")
    w = c * _NS + s
    pltpu.sync_copy(zeros_hbm.at[pl.ds(s * _RPT, _RPT)],
                    acc_sh.at[pl.ds(s * _RPT, _RPT)])
    pltpu.sync_copy(srcr_hbm.at[w], src_v)
    pltpu.sync_copy(dstr_hbm.at[w], dst_v)
    plsc.subcore_barrier()

    def body(j, carry):
        pltpu.async_copy(hs_hbm.at[src_v.at[j]], rows_v, sem).wait()
        pltpu.sync_copy(rows_v, acc_sh.at[dst_v.at[j]], add=True)
        return carry

    lax.fori_loop(0, _NCH, body, 0)
    plsc.subcore_barrier()
    pltpu.sync_copy(acc_sh.at[pl.ds(s * _RPT, _RPT)],
                    out_hbm.at[c, pl.ds(s * _RPT, _RPT)])


_RB = 1000


def _m1_body(x_ref, w_ref, dinv_ref, o_ref):
    h = jnp.dot(x_ref[...], w_ref[...], preferred_element_type=jnp.float32)
    o_ref[...] = dinv_ref[...] * h


def _m1(x, W1, dinv2):
    return pl.pallas_call(
        _m1_body,
        grid=(_N // _RB,),
        in_specs=[
            pl.BlockSpec((_RB, _D), lambda i: (i, 0)),
            pl.BlockSpec((_D, _D), lambda i: (0, 0)),
            pl.BlockSpec((_RB, 1), lambda i: (i, 0)),
        ],
        out_specs=pl.BlockSpec((_RB, _D), lambda i: (i, 0)),
        out_shape=jax.ShapeDtypeStruct((_N, _D), jnp.float32),
    )(x, W1, dinv2)


def _m2_body(a_ref, hs_ref, dinv_ref, b_ref, w_ref, o_ref):
    t = dinv_ref[...] * (a_ref[0] + a_ref[1] + hs_ref[...]) + b_ref[...]
    h = jnp.maximum(t, 0.0)
    o_ref[...] = dinv_ref[...] * jnp.dot(
        h, w_ref[...], preferred_element_type=jnp.float32)


def _m2(agg, hs, dinv2, b, W):
    return pl.pallas_call(
        _m2_body,
        grid=(_N // _RB,),
        in_specs=[
            pl.BlockSpec((2, _RB, _D), lambda i: (0, i, 0)),
            pl.BlockSpec((_RB, _D), lambda i: (i, 0)),
            pl.BlockSpec((_RB, 1), lambda i: (i, 0)),
            pl.BlockSpec((1, _D), lambda i: (0, 0)),
            pl.BlockSpec((_D, _D), lambda i: (0, 0)),
        ],
        out_specs=pl.BlockSpec((_RB, _D), lambda i: (i, 0)),
        out_shape=jax.ShapeDtypeStruct((_N, _D), jnp.float32),
    )(agg, hs, dinv2, b, W)


def _m3_body(a_ref, hs_ref, dinv_ref, b_ref, o_ref):
    t = dinv_ref[...] * (a_ref[0] + a_ref[1] + hs_ref[...]) + b_ref[...]
    o_ref[...] = dinv_ref[...] * jnp.maximum(t, 0.0)


def _m3(agg, hs, dinv2, b):
    return pl.pallas_call(
        _m3_body,
        grid=(_N // _RB,),
        in_specs=[
            pl.BlockSpec((2, _RB, _D), lambda i: (0, i, 0)),
            pl.BlockSpec((_RB, _D), lambda i: (i, 0)),
            pl.BlockSpec((_RB, 1), lambda i: (i, 0)),
            pl.BlockSpec((1, _D), lambda i: (0, 0)),
        ],
        out_specs=pl.BlockSpec((_RB, _D), lambda i: (i, 0)),
        out_shape=jax.ShapeDtypeStruct((_N, _D), jnp.float32),
    )(agg, hs, dinv2, b)


def _m4_body(a_ref, hs_ref, dinv_ref, wmu_ref, bmu_ref, wlv_ref, blv_ref,
             mu_ref, lv_ref):
    aggf = dinv_ref[...] * (a_ref[0] + a_ref[1] + hs_ref[...])
    mu_ref[...] = jnp.dot(aggf, wmu_ref[...],
                          preferred_element_type=jnp.float32) + bmu_ref[...]
    lv_ref[...] = jnp.dot(aggf, wlv_ref[...],
                          preferred_element_type=jnp.float32) + blv_ref[...]


def _m4(agg, hs, dinv2, Wmu, bmu, Wlv, blv):
    return pl.pallas_call(
        _m4_body,
        grid=(_N // _RB,),
        in_specs=[
            pl.BlockSpec((2, _RB, _D), lambda i: (0, i, 0)),
            pl.BlockSpec((_RB, _D), lambda i: (i, 0)),
            pl.BlockSpec((_RB, 1), lambda i: (i, 0)),
            pl.BlockSpec((_D, _DZ), lambda i: (0, 0)),
            pl.BlockSpec((1, _DZ), lambda i: (0, 0)),
            pl.BlockSpec((_D, _DZ), lambda i: (0, 0)),
            pl.BlockSpec((1, _DZ), lambda i: (0, 0)),
        ],
        out_specs=[
            pl.BlockSpec((_RB, _DZ), lambda i: (i, 0)),
            pl.BlockSpec((_RB, _DZ), lambda i: (i, 0)),
        ],
        out_shape=[
            jax.ShapeDtypeStruct((_N, _DZ), jnp.float32),
            jax.ShapeDtypeStruct((_N, _DZ), jnp.float32),
        ],
    )(agg, hs, dinv2, Wmu, bmu, Wlv, blv)


def kernel(x, edge_index, W1, b1, W2, b2, Wmu, bmu, Wlv, blv):
    src = edge_index[0]
    dst = edge_index[1]
    pad = _EPAD - _E
    pad_dst = _N + (jnp.arange(pad, dtype=jnp.int32) % (_NACC - _N))
    srcr = jnp.concatenate(
        [src, jnp.zeros((pad,), jnp.int32)]).reshape(_NW, _NCH, _CH)
    dstr = jnp.concatenate([dst, pad_dst]).reshape(_NW, _NCH, _CH)
    zeros1 = jnp.zeros((_NACC,), jnp.float32)
    zeros2 = jnp.zeros((_NACC, _D), jnp.float32)

    degp = _deg_kernel(dstr, zeros1).reshape(_NC, _NACC)
    deg = degp[0, :_N] + degp[1, :_N] + 1.0
    dinv2 = lax.rsqrt(deg)[:, None]

    b1r = b1[None, :]
    b2r = b2[None, :]
    bmur = bmu[None, :]
    blvr = blv[None, :]

    hs1 = _m1(x, W1, dinv2)
    agg1 = _agg_kernel(hs1, srcr, dstr, zeros2)
    hs2 = _m2(agg1[:, :_N], hs1, dinv2, b1r, W2)
    agg2 = _agg_kernel(hs2, srcr, dstr, zeros2)
    hs3 = _m3(agg2[:, :_N], hs2, dinv2, b2r)
    agg3 = _agg_kernel(hs3, srcr, dstr, zeros2)
    mu, logvar = _m4(agg3[:, :_N], hs3, dinv2, Wmu, bmur, Wlv, blvr)
    return (mu, mu, logvar)


# speedup vs baseline: 1.7015x; 1.4349x over previous
import functools

import jax
import jax.numpy as jnp
from jax import lax
from jax.experimental import pallas as pl
from jax.experimental.pallas import tpu as pltpu
from jax.experimental.pallas import tpu_sc as plsc

_N = 10000
_E = 320000
_D = 128
_DZ = 64

_NC = 2
_NS = 16
_NW = _NC * _NS
_CH = 128
_NCH = 79
_EPAD = _NW * _NCH * _CH
_RPT = 640
_NACC = _NS * _RPT


def _sc_mesh():
    return plsc.VectorSubcoreMesh(core_axis_name="c", subcore_axis_name="s")


@functools.partial(
    pl.kernel,
    mesh=_sc_mesh(),
    out_type=jax.ShapeDtypeStruct((_NC * _NACC,), jnp.float32),
    scratch_types=[
        pltpu.VMEM((_NCH, _CH), jnp.int32),
        pltpu.VMEM((_CH,), jnp.float32),
        pltpu.VMEM_SHARED((_NACC,), jnp.float32),
    ],
)
def _deg_kernel(dstr_hbm, zeros1_hbm, out_hbm, dst_v, ones_v, acc_sh):
    c = lax.axis_index("c")
    s = lax.axis_index("s")
    w = c * _NS + s
    for i in range(_CH // 16):
        ones_v[pl.ds(i * 16, 16)] = jnp.ones((16,), jnp.float32)
    pltpu.sync_copy(zeros1_hbm.at[pl.ds(s * _RPT, _RPT)],
                    acc_sh.at[pl.ds(s * _RPT, _RPT)])
    pltpu.sync_copy(dstr_hbm.at[w], dst_v)
    plsc.subcore_barrier()

    def body(j, carry):
        pltpu.sync_copy(ones_v, acc_sh.at[dst_v.at[j]], add=True)
        return carry

    lax.fori_loop(0, _NCH, body, 0)
    plsc.subcore_barrier()
    pltpu.sync_copy(acc_sh.at[pl.ds(s * _RPT, _RPT)],
                    out_hbm.at[pl.ds(c * _NACC + s * _RPT, _RPT)])


@functools.partial(
    pl.kernel,
    mesh=_sc_mesh(),
    out_type=jax.ShapeDtypeStruct((_NC, _NACC, _D), jnp.float32),
    scratch_types=[
        pltpu.VMEM((_NCH, _CH), jnp.int32),
        pltpu.VMEM((_NCH, _CH), jnp.int32),
        pltpu.VMEM((_CH, _D), jnp.float32),
        pltpu.VMEM_SHARED((_NACC, _D), jnp.float32),
        pltpu.SemaphoreType.DMA,
    ],
)
def _agg_kernel(hs_hbm, srcr_hbm, dstr_hbm, zeros_hbm, out_hbm,
                src_v, dst_v, rows_v, acc_sh, sem):
    c = lax.axis_index("c")
    s = lax.axis_index("s")
    w = c * _NS + s
    pltpu.sync_copy(zeros_hbm.at[pl.ds(s * _RPT, _RPT)],
                    acc_sh.at[pl.ds(s * _RPT, _RPT)])
    pltpu.sync_copy(srcr_hbm.at[w], src_v)
    pltpu.sync_copy(dstr_hbm.at[w], dst_v)
    plsc.subcore_barrier()

    def body(j, carry):
        pltpu.async_copy(hs_hbm.at[src_v.at[j]], rows_v, sem).wait()
        pltpu.sync_copy(rows_v, acc_sh.at[dst_v.at[j]], add=True)
        return carry

    lax.fori_loop(0, _NCH, body, 0)
    plsc.subcore_barrier()
    pltpu.sync_copy(acc_sh.at[pl.ds(s * _RPT, _RPT)],
                    out_hbm.at[c, pl.ds(s * _RPT, _RPT)])


_RB = 1000


def _m1_body(x_ref, w_ref, dinv_ref, o_ref):
    h = jnp.dot(x_ref[...], w_ref[...], preferred_element_type=jnp.float32)
    o_ref[...] = dinv_ref[...] * h


def _m1(x, W1, dinv2):
    return pl.pallas_call(
        _m1_body,
        grid=(_N // _RB,),
        in_specs=[
            pl.BlockSpec((_RB, _D), lambda i: (i, 0)),
            pl.BlockSpec((_D, _D), lambda i: (0, 0)),
            pl.BlockSpec((_RB, 1), lambda i: (i, 0)),
        ],
        out_specs=pl.BlockSpec((_RB, _D), lambda i: (i, 0)),
        out_shape=jax.ShapeDtypeStruct((_N, _D), jnp.float32),
    )(x, W1, dinv2)


def _m2_body(a_ref, hs_ref, dinv_ref, b_ref, w_ref, o_ref):
    t = dinv_ref[...] * (a_ref[0] + a_ref[1] + hs_ref[...]) + b_ref[...]
    h = jnp.maximum(t, 0.0)
    o_ref[...] = dinv_ref[...] * jnp.dot(
        h, w_ref[...], preferred_element_type=jnp.float32)


def _m2(agg, hs, dinv2, b, W):
    return pl.pallas_call(
        _m2_body,
        grid=(_N // _RB,),
        in_specs=[
            pl.BlockSpec((2, _RB, _D), lambda i: (0, i, 0)),
            pl.BlockSpec((_RB, _D), lambda i: (i, 0)),
            pl.BlockSpec((_RB, 1), lambda i: (i, 0)),
            pl.BlockSpec((1, _D), lambda i: (0, 0)),
            pl.BlockSpec((_D, _D), lambda i: (0, 0)),
        ],
        out_specs=pl.BlockSpec((_RB, _D), lambda i: (i, 0)),
        out_shape=jax.ShapeDtypeStruct((_N, _D), jnp.float32),
    )(agg, hs, dinv2, b, W)


def _m3_body(a_ref, hs_ref, dinv_ref, b_ref, o_ref):
    t = dinv_ref[...] * (a_ref[0] + a_ref[1] + hs_ref[...]) + b_ref[...]
    o_ref[...] = dinv_ref[...] * jnp.maximum(t, 0.0)


def _m3(agg, hs, dinv2, b):
    return pl.pallas_call(
        _m3_body,
        grid=(_N // _RB,),
        in_specs=[
            pl.BlockSpec((2, _RB, _D), lambda i: (0, i, 0)),
            pl.BlockSpec((_RB, _D), lambda i: (i, 0)),
            pl.BlockSpec((_RB, 1), lambda i: (i, 0)),
            pl.BlockSpec((1, _D), lambda i: (0, 0)),
        ],
        out_specs=pl.BlockSpec((_RB, _D), lambda i: (i, 0)),
        out_shape=jax.ShapeDtypeStruct((_N, _D), jnp.float32),
    )(agg, hs, dinv2, b)


def _m4_body(a_ref, hs_ref, dinv_ref, wmu_ref, bmu_ref, wlv_ref, blv_ref,
             mu_ref, lv_ref):
    aggf = dinv_ref[...] * (a_ref[0] + a_ref[1] + hs_ref[...])
    mu_ref[...] = jnp.dot(aggf, wmu_ref[...],
                          preferred_element_type=jnp.float32) + bmu_ref[...]
    lv_ref[...] = jnp.dot(aggf, wlv_ref[...],
                          preferred_element_type=jnp.float32) + blv_ref[...]


def _m4(agg, hs, dinv2, Wmu, bmu, Wlv, blv):
    return pl.pallas_call(
        _m4_body,
        grid=(_N // _RB,),
        in_specs=[
            pl.BlockSpec((2, _RB, _D), lambda i: (0, i, 0)),
            pl.BlockSpec((_RB, _D), lambda i: (i, 0)),
            pl.BlockSpec((_RB, 1), lambda i: (i, 0)),
            pl.BlockSpec((_D, _DZ), lambda i: (0, 0)),
            pl.BlockSpec((1, _DZ), lambda i: (0, 0)),
            pl.BlockSpec((_D, _DZ), lambda i: (0, 0)),
            pl.BlockSpec((1, _DZ), lambda i: (0, 0)),
        ],
        out_specs=[
            pl.BlockSpec((_RB, _DZ), lambda i: (i, 0)),
            pl.BlockSpec((_RB, _DZ), lambda i: (i, 0)),
        ],
        out_shape=[
            jax.ShapeDtypeStruct((_N, _DZ), jnp.float32),
            jax.ShapeDtypeStruct((_N, _DZ), jnp.float32),
        ],
    )(agg, hs, dinv2, Wmu, bmu, Wlv, blv)


def kernel(x, edge_index, W1, b1, W2, b2, Wmu, bmu, Wlv, blv):
    src = edge_index[0]
    dst = edge_index[1]
    pad = _EPAD - _E
    srcr = jnp.concatenate(
        [src, jnp.zeros((pad,), jnp.int32)]).reshape(_NW, _NCH, _CH)
    dstr = jnp.concatenate(
        [dst, jnp.full((pad,), _N, jnp.int32)]).reshape(_NW, _NCH, _CH)
    zeros1 = jnp.zeros((_NACC,), jnp.float32)
    zeros2 = jnp.zeros((_NACC, _D), jnp.float32)

    degp = _deg_kernel(dstr, zeros1).reshape(_NC, _NACC)
    deg = degp[0, :_N] + degp[1, :_N] + 1.0
    dinv2 = lax.rsqrt(deg)[:, None]

    b1r = b1[None, :]
    b2r = b2[None, :]
    bmur = bmu[None, :]
    blvr = blv[None, :]

    hs1 = _m1(x, W1, dinv2)
    agg1 = _agg_kernel(hs1, srcr, dstr, zeros2)
    hs2 = _m2(agg1[:, :_N], hs1, dinv2, b1r, W2)
    agg2 = _agg_kernel(hs2, srcr, dstr, zeros2)
    hs3 = _m3(agg2[:, :_N], hs2, dinv2, b2r)
    agg3 = _agg_kernel(hs3, srcr, dstr, zeros2)
    mu, logvar = _m4(agg3[:, :_N], hs3, dinv2, Wmu, bmur, Wlv, blvr)
    return (mu, mu, logvar)


# speedup vs baseline: 2.6877x; 1.5796x over previous
import functools

import jax
import jax.numpy as jnp
from jax import lax
from jax.experimental import pallas as pl
from jax.experimental.pallas import tpu as pltpu
from jax.experimental.pallas import tpu_sc as plsc

_N = 10000
_E = 320000
_D = 128
_DZ = 64

_NC = 2
_NS = 16
_NW = _NC * _NS
_CH = 128
_NCH = 79
_EPAD = _NW * _NCH * _CH
_RPT = 640
_NACC = _NS * _RPT


def _sc_mesh():
    return plsc.VectorSubcoreMesh(core_axis_name="c", subcore_axis_name="s")


@functools.partial(
    pl.kernel,
    mesh=_sc_mesh(),
    out_type=jax.ShapeDtypeStruct((_NC * _NACC,), jnp.float32),
    scratch_types=[
        pltpu.VMEM((_NCH, _CH), jnp.int32),
        pltpu.VMEM((_CH,), jnp.float32),
        pltpu.VMEM_SHARED((_NACC,), jnp.float32),
    ],
)
def _deg_kernel(dstr_hbm, zeros1_hbm, out_hbm, dst_v, ones_v, acc_sh):
    c = lax.axis_index("c")
    s = lax.axis_index("s")
    w = c * _NS + s
    for i in range(_CH // 16):
        ones_v[pl.ds(i * 16, 16)] = jnp.ones((16,), jnp.float32)
    pltpu.sync_copy(zeros1_hbm.at[pl.ds(s * _RPT, _RPT)],
                    acc_sh.at[pl.ds(s * _RPT, _RPT)])
    pltpu.sync_copy(dstr_hbm.at[w], dst_v)
    plsc.subcore_barrier()

    def body(j, carry):
        pltpu.sync_copy(ones_v, acc_sh.at[dst_v.at[j]], add=True)
        return carry

    lax.fori_loop(0, _NCH, body, 0)
    plsc.subcore_barrier()
    pltpu.sync_copy(acc_sh.at[pl.ds(s * _RPT, _RPT)],
                    out_hbm.at[pl.ds(c * _NACC + s * _RPT, _RPT)])


@functools.partial(
    pl.kernel,
    mesh=_sc_mesh(),
    out_type=jax.ShapeDtypeStruct((_NC, _NACC, _D), jnp.float32),
    scratch_types=[
        pltpu.VMEM((_NCH, _CH), jnp.int32),
        pltpu.VMEM((_NCH, _CH), jnp.int32),
        pltpu.VMEM((_CH, _D), jnp.float32),
        pltpu.VMEM_SHARED((_NACC, _D), jnp.float32),
        pltpu.SemaphoreType.DMA,
    ],
)
def _agg_kernel(hs_hbm, srcr_hbm, dstr_hbm, zeros_hbm, out_hbm,
                src_v, dst_v, rows_v, acc_sh, sem):
    c = lax.axis_index("c")
    s = lax.axis_index("s")
    w = c * _NS + s
    pltpu.sync_copy(zeros_hbm.at[pl.ds(s * _RPT, _RPT)],
                    acc_sh.at[pl.ds(s * _RPT, _RPT)])
    pltpu.sync_copy(srcr_hbm.at[w], src_v)
    pltpu.sync_copy(dstr_hbm.at[w], dst_v)
    plsc.subcore_barrier()

    def body(j, carry):
        pltpu.async_copy(hs_hbm.at[src_v.at[j]], rows_v, sem).wait()
        pltpu.sync_copy(rows_v, acc_sh.at[dst_v.at[j]], add=True)
        return carry

    lax.fori_loop(0, _NCH, body, 0)
    plsc.subcore_barrier()
    pltpu.sync_copy(acc_sh.at[pl.ds(s * _RPT, _RPT)],
                    out_hbm.at[c, pl.ds(s * _RPT, _RPT)])


_RB = 1000


def _m1_body(x_ref, w_ref, dinv_ref, o_ref):
    h = jnp.dot(x_ref[...], w_ref[...], preferred_element_type=jnp.float32)
    o_ref[...] = dinv_ref[...] * h


def _m1(x, W1, dinv2):
    return pl.pallas_call(
        _m1_body,
        grid=(_N // _RB,),
        in_specs=[
            pl.BlockSpec((_RB, _D), lambda i: (i, 0)),
            pl.BlockSpec((_D, _D), lambda i: (0, 0)),
            pl.BlockSpec((_RB, 1), lambda i: (i, 0)),
        ],
        out_specs=pl.BlockSpec((_RB, _D), lambda i: (i, 0)),
        out_shape=jax.ShapeDtypeStruct((_N, _D), jnp.float32),
    )(x, W1, dinv2)


def _m2_body(a_ref, hs_ref, dinv_ref, b_ref, w_ref, o_ref):
    t = dinv_ref[...] * (a_ref[0] + a_ref[1] + hs_ref[...]) + b_ref[...]
    h = jnp.maximum(t, 0.0)
    o_ref[...] = dinv_ref[...] * jnp.dot(
        h, w_ref[...], preferred_element_type=jnp.float32)


def _m2(agg, hs, dinv2, b, W):
    return pl.pallas_call(
        _m2_body,
        grid=(_N // _RB,),
        in_specs=[
            pl.BlockSpec((2, _RB, _D), lambda i: (0, i, 0)),
            pl.BlockSpec((_RB, _D), lambda i: (i, 0)),
            pl.BlockSpec((_RB, 1), lambda i: (i, 0)),
            pl.BlockSpec((1, _D), lambda i: (0, 0)),
            pl.BlockSpec((_D, _D), lambda i: (0, 0)),
        ],
        out_specs=pl.BlockSpec((_RB, _D), lambda i: (i, 0)),
        out_shape=jax.ShapeDtypeStruct((_N, _D), jnp.float32),
    )(agg, hs, dinv2, b, W)


def _m3_body(a_ref, hs_ref, dinv_ref, b_ref, o_ref):
    t = dinv_ref[...] * (a_ref[0] + a_ref[1] + hs_ref[...]) + b_ref[...]
    o_ref[...] = dinv_ref[...] * jnp.maximum(t, 0.0)


def _m3(agg, hs, dinv2, b):
    return pl.pallas_call(
        _m3_body,
        grid=(_N // _RB,),
        in_specs=[
            pl.BlockSpec((2, _RB, _D), lambda i: (0, i, 0)),
            pl.BlockSpec((_RB, _D), lambda i: (i, 0)),
            pl.BlockSpec((_RB, 1), lambda i: (i, 0)),
            pl.BlockSpec((1, _D), lambda i: (0, 0)),
        ],
        out_specs=pl.BlockSpec((_RB, _D), lambda i: (i, 0)),
        out_shape=jax.ShapeDtypeStruct((_N, _D), jnp.float32),
    )(agg, hs, dinv2, b)


def _m4_body(a_ref, hs_ref, dinv_ref, wmu_ref, bmu_ref, wlv_ref, blv_ref,
             mu_ref, lv_ref):
    aggf = dinv_ref[...] * (a_ref[0] + a_ref[1] + hs_ref[...])
    mu_ref[...] = jnp.dot(aggf, wmu_ref[...],
                          preferred_element_type=jnp.float32) + bmu_ref[...]
    lv_ref[...] = jnp.dot(aggf, wlv_ref[...],
                          preferred_element_type=jnp.float32) + blv_ref[...]


def _m4(agg, hs, dinv2, Wmu, bmu, Wlv, blv):
    return pl.pallas_call(
        _m4_body,
        grid=(_N // _RB,),
        in_specs=[
            pl.BlockSpec((2, _RB, _D), lambda i: (0, i, 0)),
            pl.BlockSpec((_RB, _D), lambda i: (i, 0)),
            pl.BlockSpec((_RB, 1), lambda i: (i, 0)),
            pl.BlockSpec((_D, _DZ), lambda i: (0, 0)),
            pl.BlockSpec((1, _DZ), lambda i: (0, 0)),
            pl.BlockSpec((_D, _DZ), lambda i: (0, 0)),
            pl.BlockSpec((1, _DZ), lambda i: (0, 0)),
        ],
        out_specs=[
            pl.BlockSpec((_RB, _DZ), lambda i: (i, 0)),
            pl.BlockSpec((_RB, _DZ), lambda i: (i, 0)),
        ],
        out_shape=[
            jax.ShapeDtypeStruct((_N, _DZ), jnp.float32),
            jax.ShapeDtypeStruct((_N, _DZ), jnp.float32),
        ],
    )(agg, hs, dinv2, Wmu, bmu, Wlv, blv)


def kernel(x, edge_index, W1, b1, W2, b2, Wmu, bmu, Wlv, blv):
    src = edge_index[0]
    dst = edge_index[1]
    pad = _EPAD - _E
    ar = jnp.arange(pad, dtype=jnp.int32)
    pad_src = (ar * 37) % _N
    pad_dst = _N + ar % (_NACC - _N)
    srcr = jnp.concatenate([src, pad_src]).reshape(_NW, _NCH, _CH)
    dstr = jnp.concatenate([dst, pad_dst]).reshape(_NW, _NCH, _CH)
    zeros1 = jnp.zeros((_NACC,), jnp.float32)
    zeros2 = jnp.zeros((_NACC, _D), jnp.float32)

    degp = _deg_kernel(dstr, zeros1).reshape(_NC, _NACC)
    deg = degp[0, :_N] + degp[1, :_N] + 1.0
    dinv2 = lax.rsqrt(deg)[:, None]

    b1r = b1[None, :]
    b2r = b2[None, :]
    bmur = bmu[None, :]
    blvr = blv[None, :]

    hs1 = _m1(x, W1, dinv2)
    agg1 = _agg_kernel(hs1, srcr, dstr, zeros2)
    hs2 = _m2(agg1[:, :_N], hs1, dinv2, b1r, W2)
    agg2 = _agg_kernel(hs2, srcr, dstr, zeros2)
    hs3 = _m3(agg2[:, :_N], hs2, dinv2, b2r)
    agg3 = _agg_kernel(hs3, srcr, dstr, zeros2)
    mu, logvar = _m4(agg3[:, :_N], hs3, dinv2, Wmu, bmur, Wlv, blvr)
    return (mu, mu, logvar)
